# baseline (speedup 1.0000x reference)
import functools

import jax
import jax.numpy as jnp
from jax import lax
from jax.experimental import pallas as pl
from jax.experimental.pallas import tpu as pltpu
from jax.experimental.pallas import tpu_sc as plsc

BATCH = 16384
DIM = 64

_info = plsc.get_sparse_core_info()
NC, NS, L = _info.num_cores, _info.num_subcores, _info.num_lanes
NW = NC * NS
BPW = BATCH // NW
CHUNK = 128
NCHUNK = BPW // CHUNK

_mesh = plsc.VectorSubcoreMesh(core_axis_name="c", subcore_axis_name="s")


@functools.partial(
    pl.kernel,
    mesh=_mesh,
    compiler_params=pltpu.CompilerParams(use_tc_tiling_on_sc=False),
    out_type=jax.ShapeDtypeStruct((BATCH, DIM), jnp.float32),
    scratch_types=[
        pltpu.VMEM((NCHUNK, CHUNK), jnp.int32),
        pltpu.VMEM((BPW, DIM), jnp.float32),
        pltpu.VMEM((BPW, DIM), jnp.float32),
        pltpu.VMEM((BPW, DIM), jnp.float32),
        pltpu.SemaphoreType.DMA,
    ],
)
def _calibrate(x_hbm, u_hbm, scale_hbm, bias_hbm, out_hbm,
               idx_v, x_v, s_v, b_v, sem):
    wid = lax.axis_index("s") * NC + lax.axis_index("c")
    base = wid * BPW

    pltpu.sync_copy(u_hbm.at[wid], idx_v)

    copies = []
    for c in range(NCHUNK):
        rows = pl.ds(c * CHUNK, CHUNK)
        copies.append(pltpu.async_copy(scale_hbm.at[idx_v.at[c]], s_v.at[rows], sem))
        copies.append(pltpu.async_copy(bias_hbm.at[idx_v.at[c]], b_v.at[rows], sem))
    pltpu.sync_copy(x_hbm.at[pl.ds(base, BPW)], x_v)
    for cp in copies:
        cp.wait()

    def row(r, carry):
        for j in range(DIM // L):
            sl = pl.ds(j * L, L)
            x_v[r, sl] = x_v[r, sl] * s_v[r, sl] + b_v[r, sl]
        return carry

    lax.fori_loop(0, BPW, row, 0)

    pltpu.sync_copy(x_v, out_hbm.at[pl.ds(base, BPW)])


def kernel(x, u, scale_weight, bias_weight):
    u3 = u.reshape(NW, NCHUNK, CHUNK)
    return _calibrate(x, u3, scale_weight, bias_weight)


# speedup vs baseline: 1.5687x; 1.5687x over previous
import functools

import jax
import jax.numpy as jnp
from jax import lax
from jax.experimental import pallas as pl
from jax.experimental.pallas import tpu as pltpu
from jax.experimental.pallas import tpu_sc as plsc

BATCH = 16384
DIM = 64
NUSERS = 1000000

_info = plsc.get_sparse_core_info()
NC, NS, L = _info.num_cores, _info.num_subcores, _info.num_lanes
NW = NC * NS
BPW = BATCH // NW
NBLK = (NUSERS + 127) // 128
NBPT = (NBLK + NW - 1) // NW
CHB = 2
CHU = CHB * 128
NVR = BATCH // L
PAD = BATCH

_mesh = plsc.VectorSubcoreMesh(core_axis_name="c", subcore_axis_name="s")


@functools.partial(
    pl.kernel,
    mesh=_mesh,
    compiler_params=pltpu.CompilerParams(use_tc_tiling_on_sc=True, needs_layout_passes=False),
    out_type=jax.ShapeDtypeStruct((BATCH + NW, 2 * DIM), jnp.float32),
    scratch_types=[
        pltpu.VMEM((BATCH,), jnp.int32),
        pltpu.VMEM((BATCH + L,), jnp.int32),
        pltpu.VMEM((BATCH + L,), jnp.int32),
        pltpu.VMEM((DIM, CHU), jnp.float32),
        pltpu.VMEM((DIM, CHU), jnp.float32),
        pltpu.VMEM((3 * L,), jnp.int32),
        pltpu.VMEM((3 * L,), jnp.int32),
        pltpu.VMEM((1, L), jnp.int32),
        pltpu.VMEM((L, 2 * DIM), jnp.float32),
    ],
)
def _gather_phase(u_hbm, sT_hbm, bT_hbm, sb_hbm,
                  u_v, wl_e, wl_u, s_ch, b_ch, wv_e, wv_c, idx2, rsb_v):
    wid = lax.axis_index("s") * NC + lax.axis_index("c")
    lane = lax.iota(jnp.int32, L)
    lo = wid * NBPT
    hi = jnp.minimum(lo + NBPT, NBLK)

    pltpu.sync_copy(u_hbm, u_v)

    def scan(ci, cnt):
        uvec = u_v[pl.ds(ci * L, L)]
        blk = uvec >> 7
        m = (blk >= lo) & (blk < hi)
        pos = cnt + plsc.cumsum(jnp.where(m, 1, 0)) - 1
        plsc.store_scatter(wl_e, [pos], ci * L + lane, mask=m)
        plsc.store_scatter(wl_u, [pos], uvec, mask=m)
        return cnt + jnp.sum(jnp.where(m, 1, 0))

    cnt = lax.fori_loop(0, NVR, scan, jnp.int32(0))
    nl = (cnt + L - 1) // L

    def flush(evec, cvec):
        idx2[0, :] = evec
        for e in range(L):
            ce = jnp.sum(jnp.where(lane == e, cvec, 0))
            cs = jnp.broadcast_to(ce, (L,)).astype(jnp.int32)
            for j in range(DIM // L):
                rvec = j * L + lane
                sl = pl.ds(j * L, L)
                rsb_v[e, sl] = plsc.load_gather(s_ch, [rvec, cs])
                rsb_v[e, pl.ds(DIM + j * L, L)] = plsc.load_gather(b_ch, [rvec, cs])
        pltpu.sync_copy(rsb_v, sb_hbm.at[idx2.at[0]])

    def chunk(q, _):
        cblk = lo + q * CHB
        off = jnp.minimum(cblk * 128, NUSERS - CHU)
        offm = pl.multiple_of(off, 128)
        pltpu.sync_copy(sT_hbm.at[:, pl.ds(offm, CHU)], s_ch)
        pltpu.sync_copy(bT_hbm.at[:, pl.ds(offm, CHU)], b_ch)
        bhi = jnp.minimum(cblk + CHB, hi)

        def rescan(ci, wcnt):
            uvec = wl_u[pl.ds(ci * L, L)]
            evec = wl_e[pl.ds(ci * L, L)]
            blk = uvec >> 7
            valid = (ci * L + lane) < cnt
            m = (blk >= cblk) & (blk < bhi) & valid
            pos = wcnt + plsc.cumsum(jnp.where(m, 1, 0)) - 1
            plsc.store_scatter(wv_e, [pos], evec, mask=m)
            plsc.store_scatter(wv_c, [pos], uvec - off, mask=m)
            wcnt = wcnt + jnp.sum(jnp.where(m, 1, 0))

            def do_flush(w):
                flush(wv_e[pl.ds(0, L)], wv_c[pl.ds(0, L)])
                se = plsc.load_gather(wv_e, [L + lane])
                sc = plsc.load_gather(wv_c, [L + lane])
                wv_e[pl.ds(0, L)] = se
                wv_c[pl.ds(0, L)] = sc
                return w - L

            return lax.cond(wcnt >= L, do_flush, lambda w: w, wcnt)

        wcnt = lax.fori_loop(0, nl, rescan, jnp.int32(0))

        def tail_flush(w):
            evec = jnp.where(lane < w, wv_e[pl.ds(0, L)], PAD + wid)
            cvec = jnp.where(lane < w, wv_c[pl.ds(0, L)], 0)
            flush(evec, cvec)
            return 0

        lax.cond(wcnt > 0, tail_flush, lambda w: 0, wcnt)
        return 0

    nch = (hi - lo + CHB - 1) // CHB
    lax.fori_loop(0, nch, chunk, 0)


@functools.partial(
    pl.kernel,
    mesh=_mesh,
    compiler_params=pltpu.CompilerParams(use_tc_tiling_on_sc=False),
    out_type=jax.ShapeDtypeStruct((BATCH, DIM), jnp.float32),
    scratch_types=[
        pltpu.VMEM((BPW, DIM), jnp.float32),
        pltpu.VMEM((BPW, 2 * DIM), jnp.float32),
        pltpu.SemaphoreType.DMA,
    ],
)
def _apply_phase(x_hbm, sb_hbm, out_hbm, x_v, sb_v, sem):
    wid = lax.axis_index("s") * NC + lax.axis_index("c")
    base = wid * BPW
    rows = pl.ds(base, BPW)
    c1 = pltpu.async_copy(sb_hbm.at[rows], sb_v, sem)
    pltpu.sync_copy(x_hbm.at[rows], x_v)
    c1.wait()

    def row(r, carry):
        for j in range(DIM // L):
            sl = pl.ds(j * L, L)
            x_v[r, sl] = x_v[r, sl] * sb_v[r, sl] + sb_v[r, pl.ds(DIM + j * L, L)]
        return carry

    lax.fori_loop(0, BPW, row, 0)
    pltpu.sync_copy(x_v, out_hbm.at[rows])


def kernel(x, u, scale_weight, bias_weight):
    sb = _gather_phase(u, scale_weight.T, bias_weight.T)
    return _apply_phase(x, sb)


# speedup vs baseline: 2.6344x; 1.6793x over previous
import functools

import jax
import jax.numpy as jnp
from jax import lax
from jax.experimental import pallas as pl
from jax.experimental.pallas import tpu as pltpu
from jax.experimental.pallas import tpu_sc as plsc

BATCH = 16384
DIM = 64
NUSERS = 1000000

_info = plsc.get_sparse_core_info()
NC, NS, L = _info.num_cores, _info.num_subcores, _info.num_lanes
NW = NC * NS
BPW = BATCH // NW
NBLK = (NUSERS + 127) // 128
NBPT = (NBLK + NW - 1) // NW
CHB = 2
CHU = CHB * 128
NVR = BATCH // L
PAD = BATCH

_mesh = plsc.VectorSubcoreMesh(core_axis_name="c", subcore_axis_name="s")


@functools.partial(
    pl.kernel,
    mesh=_mesh,
    compiler_params=pltpu.CompilerParams(use_tc_tiling_on_sc=True, needs_layout_passes=False),
    out_type=jax.ShapeDtypeStruct((BATCH + NW, 2 * DIM), jnp.float32),
    scratch_types=[
        pltpu.VMEM((BATCH,), jnp.int32),
        pltpu.VMEM((BATCH + L,), jnp.int32),
        pltpu.VMEM((BATCH + L,), jnp.int32),
        pltpu.VMEM((2, DIM, CHU), jnp.float32),
        pltpu.VMEM((2, DIM, CHU), jnp.float32),
        pltpu.VMEM((3 * L,), jnp.int32),
        pltpu.VMEM((3 * L,), jnp.int32),
        pltpu.VMEM((2, 1, L), jnp.int32),
        pltpu.VMEM((2, L, 2 * DIM), jnp.float32),
        pltpu.SemaphoreType.DMA,
        pltpu.SemaphoreType.DMA,
    ],
)
def _gather_phase(u_hbm, sT_hbm, bT_hbm, sb_hbm,
                  u_v, wl_e, wl_u, s_ch, b_ch, wv_e, wv_c, idx2, rsb_v,
                  sem, sem2):
    wid = lax.axis_index("s") * NC + lax.axis_index("c")
    lane = lax.iota(jnp.int32, L)
    lo = wid * NBPT
    hi = jnp.minimum(lo + NBPT, NBLK)

    pltpu.sync_copy(u_hbm, u_v)

    def scan(ci, cnt):
        uvec = u_v[pl.ds(ci * L, L)]
        blk = uvec >> 7
        m = (blk >= lo) & (blk < hi)
        pos = cnt + plsc.cumsum(jnp.where(m, 1, 0)) - 1
        plsc.store_scatter(wl_e, [pos], ci * L + lane, mask=m)
        plsc.store_scatter(wl_u, [pos], uvec, mask=m)
        return cnt + jnp.sum(jnp.where(m, 1, 0))

    cnt = lax.fori_loop(0, NVR, scan, jnp.int32(0))
    nl = (cnt + L - 1) // L

    def chunk_off(q):
        return pl.multiple_of(jnp.minimum((lo + q * CHB) * 128, NUSERS - CHU), 128)

    def issue(q):
        b = q & 1
        offm = chunk_off(q)
        pltpu.async_copy(sT_hbm.at[:, pl.ds(offm, CHU)], s_ch.at[b], sem)
        pltpu.async_copy(bT_hbm.at[:, pl.ds(offm, CHU)], b_ch.at[b], sem)

    def flush(evec, cvec, fl, b):
        slot = fl & 1

        @pl.when(fl >= 2)
        def _():
            pltpu.make_async_copy(
                sb_hbm.at[pl.ds(0, L)], rsb_v.at[0], sem2).wait()

        idx2[slot, 0, :] = evec
        for e in range(L):
            ce = jnp.sum(jnp.where(lane == e, cvec, 0))
            cs = jnp.broadcast_to(ce, (L,)).astype(jnp.int32)
            for j in range(DIM // L):
                rvec = j * L + lane
                rsb_v[slot, e, pl.ds(j * L, L)] = plsc.load_gather(
                    s_ch.at[b], [rvec, cs])
                rsb_v[slot, e, pl.ds(DIM + j * L, L)] = plsc.load_gather(
                    b_ch.at[b], [rvec, cs])
        pltpu.async_copy(rsb_v.at[slot], sb_hbm.at[idx2.at[slot].at[0]], sem2)
        return fl + 1

    nch = (hi - lo + CHB - 1) // CHB
    issue(0)

    def chunk(q, fl):
        b = q & 1
        offm = chunk_off(q)
        off = jnp.minimum((lo + q * CHB) * 128, NUSERS - CHU)
        pltpu.make_async_copy(sT_hbm.at[:, pl.ds(offm, CHU)], s_ch.at[b], sem).wait()
        pltpu.make_async_copy(bT_hbm.at[:, pl.ds(offm, CHU)], b_ch.at[b], sem).wait()

        @pl.when(q + 1 < nch)
        def _():
            issue(q + 1)

        cblk = lo + q * CHB
        bhi = jnp.minimum(cblk + CHB, hi)

        def rescan(ci, carry):
            wcnt, fl = carry
            uvec = wl_u[pl.ds(ci * L, L)]
            evec = wl_e[pl.ds(ci * L, L)]
            blk = uvec >> 7
            valid = (ci * L + lane) < cnt
            m = (blk >= cblk) & (blk < bhi) & valid
            pos = wcnt + plsc.cumsum(jnp.where(m, 1, 0)) - 1
            plsc.store_scatter(wv_e, [pos], evec, mask=m)
            plsc.store_scatter(wv_c, [pos], uvec - off, mask=m)
            wcnt = wcnt + jnp.sum(jnp.where(m, 1, 0))

            def do_flush(t):
                w, f = t
                f = flush(wv_e[pl.ds(0, L)], wv_c[pl.ds(0, L)], f, b)
                se = plsc.load_gather(wv_e, [L + lane])
                sc = plsc.load_gather(wv_c, [L + lane])
                wv_e[pl.ds(0, L)] = se
                wv_c[pl.ds(0, L)] = sc
                return (w - L, f)

            return lax.cond(wcnt >= L, do_flush, lambda t: t, (wcnt, fl))

        wcnt, fl = lax.fori_loop(0, nl, rescan, (jnp.int32(0), fl))

        def tail_flush(t):
            w, f = t
            evec = jnp.where(lane < w, wv_e[pl.ds(0, L)], PAD + wid)
            cvec = jnp.where(lane < w, wv_c[pl.ds(0, L)], 0)
            return flush(evec, cvec, f, b)

        fl = lax.cond(wcnt > 0, tail_flush, lambda t: t[1], (wcnt, fl))
        return fl

    fl = lax.fori_loop(0, nch, chunk, jnp.int32(0))

    def drain(i, c):
        pltpu.make_async_copy(sb_hbm.at[pl.ds(0, L)], rsb_v.at[0], sem2).wait()
        return c

    lax.fori_loop(0, jnp.minimum(fl, 2), drain, 0)


@functools.partial(
    pl.kernel,
    mesh=_mesh,
    compiler_params=pltpu.CompilerParams(use_tc_tiling_on_sc=False),
    out_type=jax.ShapeDtypeStruct((BATCH, DIM), jnp.float32),
    scratch_types=[
        pltpu.VMEM((BPW, DIM), jnp.float32),
        pltpu.VMEM((BPW, 2 * DIM), jnp.float32),
        pltpu.SemaphoreType.DMA,
    ],
)
def _apply_phase(x_hbm, sb_hbm, out_hbm, x_v, sb_v, sem):
    wid = lax.axis_index("s") * NC + lax.axis_index("c")
    base = wid * BPW
    rows = pl.ds(base, BPW)
    c1 = pltpu.async_copy(sb_hbm.at[rows], sb_v, sem)
    pltpu.sync_copy(x_hbm.at[rows], x_v)
    c1.wait()

    def row(r, carry):
        for j in range(DIM // L):
            sl = pl.ds(j * L, L)
            x_v[r, sl] = x_v[r, sl] * sb_v[r, sl] + sb_v[r, pl.ds(DIM + j * L, L)]
        return carry

    lax.fori_loop(0, BPW, row, 0)
    pltpu.sync_copy(x_v, out_hbm.at[rows])


def kernel(x, u, scale_weight, bias_weight):
    sb = _gather_phase(u, scale_weight.T, bias_weight.T)
    return _apply_phase(x, sb)


# speedup vs baseline: 2.6516x; 1.0065x over previous
import functools

import jax
import jax.numpy as jnp
from jax import lax
from jax.experimental import pallas as pl
from jax.experimental.pallas import tpu as pltpu
from jax.experimental.pallas import tpu_sc as plsc

BATCH = 16384
DIM = 64
NUSERS = 1000000

_info = plsc.get_sparse_core_info()
NC, NS, L = _info.num_cores, _info.num_subcores, _info.num_lanes
NW = NC * NS
BPW = BATCH // NW
NBLK = (NUSERS + 127) // 128
NBPT = (NBLK + NW - 1) // NW
CHB = 2
CHU = CHB * 128
NVR = BATCH // L
PAD = BATCH

_mesh = plsc.VectorSubcoreMesh(core_axis_name="c", subcore_axis_name="s")


@functools.partial(
    pl.kernel,
    mesh=_mesh,
    compiler_params=pltpu.CompilerParams(use_tc_tiling_on_sc=True, needs_layout_passes=False),
    out_type=jax.ShapeDtypeStruct((BATCH + NW, 2 * DIM), jnp.float32),
    scratch_types=[
        pltpu.VMEM((BATCH,), jnp.int32),
        pltpu.VMEM((BATCH + L,), jnp.int32),
        pltpu.VMEM((BATCH + L,), jnp.int32),
        pltpu.VMEM((2, DIM, CHU), jnp.float32),
        pltpu.VMEM((2, DIM, CHU), jnp.float32),
        pltpu.VMEM((3 * L,), jnp.int32),
        pltpu.VMEM((3 * L,), jnp.int32),
        pltpu.VMEM((4, 1, L), jnp.int32),
        pltpu.VMEM((4, L, 2 * DIM), jnp.float32),
        pltpu.SemaphoreType.DMA,
        pltpu.SemaphoreType.DMA,
    ],
)
def _gather_phase(u_hbm, sT_hbm, bT_hbm, sb_hbm,
                  u_v, wl_e, wl_u, s_ch, b_ch, wv_e, wv_c, idx2, rsb_v,
                  sem, sem2):
    wid = lax.axis_index("s") * NC + lax.axis_index("c")
    lane = lax.iota(jnp.int32, L)
    lo = wid * NBPT
    hi = jnp.minimum(lo + NBPT, NBLK)

    pltpu.sync_copy(u_hbm, u_v)

    def scan(ci, cnt):
        uvec = u_v[pl.ds(ci * L, L)]
        blk = uvec >> 7
        m = (blk >= lo) & (blk < hi)
        pos = cnt + plsc.cumsum(jnp.where(m, 1, 0)) - 1
        plsc.store_scatter(wl_e, [pos], ci * L + lane, mask=m)
        plsc.store_scatter(wl_u, [pos], uvec, mask=m)
        return cnt + jnp.sum(jnp.where(m, 1, 0))

    cnt = lax.fori_loop(0, NVR, scan, jnp.int32(0))
    nl = (cnt + L - 1) // L

    def chunk_off(q):
        return pl.multiple_of(jnp.minimum((lo + q * CHB) * 128, NUSERS - CHU), 128)

    def issue(q):
        b = q & 1
        offm = chunk_off(q)
        pltpu.async_copy(sT_hbm.at[:, pl.ds(offm, CHU)], s_ch.at[b], sem)
        pltpu.async_copy(bT_hbm.at[:, pl.ds(offm, CHU)], b_ch.at[b], sem)

    def flush(evec, cvec, fl, b):
        slot = fl & 3

        @pl.when(fl >= 4)
        def _():
            pltpu.make_async_copy(
                sb_hbm.at[pl.ds(0, L)], rsb_v.at[0], sem2).wait()

        idx2[slot, 0, :] = evec
        for e in range(L):
            ce = jnp.sum(jnp.where(lane == e, cvec, 0))
            cs = jnp.broadcast_to(ce, (L,)).astype(jnp.int32)
            for j in range(DIM // L):
                rvec = j * L + lane
                rsb_v[slot, e, pl.ds(j * L, L)] = plsc.load_gather(
                    s_ch.at[b], [rvec, cs])
                rsb_v[slot, e, pl.ds(DIM + j * L, L)] = plsc.load_gather(
                    b_ch.at[b], [rvec, cs])
        pltpu.async_copy(rsb_v.at[slot], sb_hbm.at[idx2.at[slot].at[0]], sem2)
        return fl + 1

    nch = (hi - lo + CHB - 1) // CHB
    issue(0)

    def chunk(q, fl):
        b = q & 1
        offm = chunk_off(q)
        off = jnp.minimum((lo + q * CHB) * 128, NUSERS - CHU)
        pltpu.make_async_copy(sT_hbm.at[:, pl.ds(offm, CHU)], s_ch.at[b], sem).wait()
        pltpu.make_async_copy(bT_hbm.at[:, pl.ds(offm, CHU)], b_ch.at[b], sem).wait()

        @pl.when(q + 1 < nch)
        def _():
            issue(q + 1)

        cblk = lo + q * CHB
        bhi = jnp.minimum(cblk + CHB, hi)

        def rescan(ci, carry):
            wcnt, fl = carry
            uvec = wl_u[pl.ds(ci * L, L)]
            evec = wl_e[pl.ds(ci * L, L)]
            blk = uvec >> 7
            valid = (ci * L + lane) < cnt
            m = (blk >= cblk) & (blk < bhi) & valid
            pos = wcnt + plsc.cumsum(jnp.where(m, 1, 0)) - 1
            plsc.store_scatter(wv_e, [pos], evec, mask=m)
            plsc.store_scatter(wv_c, [pos], uvec - off, mask=m)
            wcnt = wcnt + jnp.sum(jnp.where(m, 1, 0))

            def do_flush(t):
                w, f = t
                f = flush(wv_e[pl.ds(0, L)], wv_c[pl.ds(0, L)], f, b)
                se = plsc.load_gather(wv_e, [L + lane])
                sc = plsc.load_gather(wv_c, [L + lane])
                wv_e[pl.ds(0, L)] = se
                wv_c[pl.ds(0, L)] = sc
                return (w - L, f)

            return lax.cond(wcnt >= L, do_flush, lambda t: t, (wcnt, fl))

        wcnt, fl = lax.fori_loop(0, nl, rescan, (jnp.int32(0), fl))

        def tail_flush(t):
            w, f = t
            evec = jnp.where(lane < w, wv_e[pl.ds(0, L)], PAD + wid)
            cvec = jnp.where(lane < w, wv_c[pl.ds(0, L)], 0)
            return flush(evec, cvec, f, b)

        fl = lax.cond(wcnt > 0, tail_flush, lambda t: t[1], (wcnt, fl))
        return fl

    fl = lax.fori_loop(0, nch, chunk, jnp.int32(0))

    def drain(i, c):
        pltpu.make_async_copy(sb_hbm.at[pl.ds(0, L)], rsb_v.at[0], sem2).wait()
        return c

    lax.fori_loop(0, jnp.minimum(fl, 4), drain, 0)


@functools.partial(
    pl.kernel,
    mesh=_mesh,
    compiler_params=pltpu.CompilerParams(use_tc_tiling_on_sc=False),
    out_type=jax.ShapeDtypeStruct((BATCH, DIM), jnp.float32),
    scratch_types=[
        pltpu.VMEM((BPW, DIM), jnp.float32),
        pltpu.VMEM((BPW, 2 * DIM), jnp.float32),
        pltpu.SemaphoreType.DMA,
    ],
)
def _apply_phase(x_hbm, sb_hbm, out_hbm, x_v, sb_v, sem):
    wid = lax.axis_index("s") * NC + lax.axis_index("c")
    base = wid * BPW
    rows = pl.ds(base, BPW)
    c1 = pltpu.async_copy(sb_hbm.at[rows], sb_v, sem)
    pltpu.sync_copy(x_hbm.at[rows], x_v)
    c1.wait()

    def row(r, carry):
        for j in range(DIM // L):
            sl = pl.ds(j * L, L)
            x_v[r, sl] = x_v[r, sl] * sb_v[r, sl] + sb_v[r, pl.ds(DIM + j * L, L)]
        return carry

    lax.fori_loop(0, BPW, row, 0)
    pltpu.sync_copy(x_v, out_hbm.at[rows])


def kernel(x, u, scale_weight, bias_weight):
    sb = _gather_phase(u, scale_weight.T, bias_weight.T)
    return _apply_phase(x, sb)
